# baseline (device time: 136336 ns/iter reference)
import jax
import jax.numpy as jnp
from jax import lax
from jax.experimental import pallas as pl
from jax.experimental.pallas import tpu as pltpu

N_DEV = 8
SQ = 128
D = 1024
HQ = 8
HKV = 2
DH = 128
SKV = 32768
CHUNK = 1024
NCHUNKS = SKV // CHUNK
SCALE = 0.08838834764831843
ROWS = HQ * SQ
GROUP = ROWS // HKV
PACK = ROWS + HQ


def _eye128():
    i = lax.broadcasted_iota(jnp.int32, (128, 128), 0)
    j = lax.broadcasted_iota(jnp.int32, (128, 128), 1)
    return (i == j).astype(jnp.float32)


def _col_to_row(col):
    return jnp.sum(col * _eye128(), axis=0, keepdims=True)


def _row_to_col(row):
    return jnp.sum(row * _eye128(), axis=1, keepdims=True)


def kernel(x, Wq, Wo, K_ext, V_ext):
    x2 = x.reshape(SQ, D)
    k2 = K_ext.reshape(SKV, HKV * DH)
    v2 = V_ext.reshape(SKV, HKV * DH)

    def body(x_ref, wq_ref, wo_ref, k_ref, v_ref, o_ref,
             q_ref, accl_ref, lcol_ref, recv_ref, t_ref,
             send_sems, recv_sems):
        j = pl.program_id(0)

        @pl.when(j == 0)
        def _init():
            accl_ref[...] = jnp.zeros_like(accl_ref)
            lcol_ref[...] = jnp.zeros_like(lcol_ref)
            xb = x_ref[...].astype(jnp.bfloat16)
            for h in range(HQ):
                qh = jnp.dot(
                    xb, wq_ref[:, h * DH:(h + 1) * DH].astype(jnp.bfloat16),
                    preferred_element_type=jnp.float32)
                q_ref[pl.ds(h * SQ, SQ), :] = qh * SCALE

        for g in range(HKV):
            qg = q_ref[pl.ds(g * GROUP, GROUP), :].astype(jnp.bfloat16)
            kg = k_ref[:, g * DH:(g + 1) * DH].astype(jnp.bfloat16)
            vg = v_ref[:, g * DH:(g + 1) * DH].astype(jnp.bfloat16)
            s = lax.dot_general(qg, kg, (((1,), (1,)), ((), ())),
                                preferred_element_type=jnp.float32)
            p = jnp.exp(s)
            lcol_ref[pl.ds(g * GROUP, GROUP), :] += jnp.sum(
                p, axis=1, keepdims=True)
            accl_ref[pl.ds(g * GROUP, GROUP), :] += lax.dot_general(
                p.astype(jnp.bfloat16), vg, (((1,), (0,)), ((), ())),
                preferred_element_type=jnp.float32)

        @pl.when(j == NCHUNKS - 1)
        def _combine():
            me = lax.axis_index("i")

            for h in range(HQ):
                accl_ref[pl.ds(ROWS + h, 1), :] = _col_to_row(
                    lcol_ref[pl.ds(h * SQ, SQ), :])

            barrier = pltpu.get_barrier_semaphore()
            for k in range(1, N_DEV):
                pl.semaphore_signal(
                    barrier, inc=1,
                    device_id=((me + k) % N_DEV,),
                    device_id_type=pl.DeviceIdType.MESH)
            pl.semaphore_wait(barrier, N_DEV - 1)

            rdmas = []
            for k in range(1, N_DEV):
                rdma = pltpu.make_async_remote_copy(
                    src_ref=accl_ref,
                    dst_ref=recv_ref.at[k - 1],
                    send_sem=send_sems.at[k - 1],
                    recv_sem=recv_sems.at[k - 1],
                    device_id=((me + k) % N_DEV,),
                    device_id_type=pl.DeviceIdType.MESH)
                rdma.start()
                rdmas.append(rdma)
            for rdma in rdmas:
                rdma.wait_send()
            for k in range(1, N_DEV):
                rdmas[k - 1].wait_recv()
                accl_ref[...] += recv_ref[k - 1]

            for h in range(HQ):
                lh = _row_to_col(accl_ref[pl.ds(ROWS + h, 1), :])
                t_ref[:, pl.ds(h * DH, DH)] = (
                    accl_ref[pl.ds(h * SQ, SQ), :] / lh)
            o_ref[...] = jnp.dot(
                t_ref[...].astype(jnp.bfloat16),
                wo_ref[...].astype(jnp.bfloat16),
                preferred_element_type=jnp.float32)

    out = pl.pallas_call(
        body,
        grid=(NCHUNKS,),
        in_specs=[
            pl.BlockSpec((SQ, D), lambda j: (0, 0)),
            pl.BlockSpec((D, D), lambda j: (0, 0)),
            pl.BlockSpec((D, D), lambda j: (0, 0)),
            pl.BlockSpec((CHUNK, HKV * DH), lambda j: (j, 0)),
            pl.BlockSpec((CHUNK, HKV * DH), lambda j: (j, 0)),
        ],
        out_specs=pl.BlockSpec((SQ, D), lambda j: (0, 0)),
        out_shape=jax.ShapeDtypeStruct((SQ, D), jnp.float32),
        scratch_shapes=[
            pltpu.VMEM((ROWS, DH), jnp.float32),
            pltpu.VMEM((PACK, DH), jnp.float32),
            pltpu.VMEM((ROWS, 1), jnp.float32),
            pltpu.VMEM((N_DEV - 1, PACK, DH), jnp.float32),
            pltpu.VMEM((SQ, D), jnp.float32),
            pltpu.SemaphoreType.DMA((N_DEV - 1,)),
            pltpu.SemaphoreType.DMA((N_DEV - 1,)),
        ],
        compiler_params=pltpu.CompilerParams(
            dimension_semantics=("arbitrary",),
            collective_id=0,
        ),
    )(x2, Wq, Wo, k2, v2)
    return out.reshape(1, SQ, D)


# device time: 61326 ns/iter; 2.2231x vs baseline; 2.2231x over previous
import os

import jax
import jax.numpy as jnp
from jax import lax
from jax.experimental import pallas as pl
from jax.experimental.pallas import tpu as pltpu

N_DEV = 8
_COMM = not os.path.exists(os.path.join(os.path.dirname(__file__), "NO_COMM"))
SQ = 128
D = 1024
HQ = 8
HKV = 2
DH = 128
SKV = 32768
CHUNK = 4096
NCHUNKS = SKV // CHUNK
SCALE = 0.08838834764831843
ROWS = HQ * SQ
GROUP = ROWS // HKV
PACK = 1088
SLICE = PACK // N_DEV


def _eye128():
    i = lax.broadcasted_iota(jnp.int32, (128, 128), 0)
    j = lax.broadcasted_iota(jnp.int32, (128, 128), 1)
    return (i == j).astype(jnp.float32)


def _col_to_row(col):
    return jnp.sum(col * _eye128(), axis=0, keepdims=True)


def _row_to_col(row):
    return jnp.sum(row * _eye128(), axis=1, keepdims=True)


def kernel(x, Wq, Wo, K_ext, V_ext):
    x2 = x.reshape(SQ, D)
    k3 = K_ext.reshape(SKV, HKV, DH)
    v3 = V_ext.reshape(SKV, HKV, DH)

    def body(x_hbm, wq_hbm, wo_hbm, k_hbm, v_hbm, o_ref,
             q_ref, accl_ref, lcol_ref, rs_recv, ag_recv, t_ref, kv_scr,
             x_s, wq_s, wo_s,
             copy_sems, stage_sems, send_sems, recv_sems,
             send_sems2, recv_sems2):
        j = pl.program_id(0)

        stage = [
            pltpu.make_async_copy(src, dst, stage_sems.at[i])
            for i, (src, dst) in enumerate(
                ((x_hbm, x_s), (wq_hbm, wq_s), (wo_hbm, wo_s)))
        ]

        def kv_copies(c, slot):
            out = []
            for t, (hbm, g) in enumerate(
                    ((k_hbm, 0), (k_hbm, 1), (v_hbm, 0), (v_hbm, 1))):
                out.append(pltpu.make_async_copy(
                    hbm.at[pl.ds(c * CHUNK, CHUNK), g],
                    kv_scr.at[slot, t],
                    copy_sems.at[slot, t]))
            return out

        @pl.when(j == 0)
        def _init():
            for cp in stage:
                cp.start()
            for cp in kv_copies(0, 0):
                cp.start()
            for cp in kv_copies(1, 1):
                cp.start()
            accl_ref[...] = jnp.zeros_like(accl_ref)
            lcol_ref[...] = jnp.zeros_like(lcol_ref)
            stage[0].wait()
            stage[1].wait()
            xb = x_s[...].astype(jnp.bfloat16)
            for h in range(HQ):
                qh = jnp.dot(
                    xb, wq_s[:, h * DH:(h + 1) * DH].astype(jnp.bfloat16),
                    preferred_element_type=jnp.float32)
                q_ref[pl.ds(h * SQ, SQ), :] = qh * (SCALE * 1.4426950408889634)

        @pl.when(jnp.logical_and(j > 0, j < NCHUNKS - 1))
        def _prefetch():
            for cp in kv_copies(j + 1, (j + 1) % 2):
                cp.start()

        slot = j % 2
        for cp in kv_copies(j, slot):
            cp.wait()

        for g in range(HKV):
            qg = q_ref[pl.ds(g * GROUP, GROUP), :].astype(jnp.bfloat16)
            kg = kv_scr[slot, g].astype(jnp.bfloat16)
            vg = kv_scr[slot, 2 + g].astype(jnp.bfloat16)
            s = lax.dot_general(qg, kg, (((1,), (1,)), ((), ())),
                                preferred_element_type=jnp.float32)
            p = jnp.exp2(s)
            lcol_ref[pl.ds(g * GROUP, GROUP), :] += jnp.sum(
                p, axis=1, keepdims=True)
            accl_ref[pl.ds(g * GROUP, GROUP), :] += lax.dot_general(
                p.astype(jnp.bfloat16), vg, (((1,), (0,)), ((), ())),
                preferred_element_type=jnp.float32)

        @pl.when(j == NCHUNKS - 1)
        def _combine():
            me = lax.axis_index("i")

            for h in range(HQ):
                accl_ref[pl.ds(ROWS + h, 1), :] = _col_to_row(
                    lcol_ref[pl.ds(h * SQ, SQ), :])

            if not _COMM:
                return

            barrier = pltpu.get_barrier_semaphore()
            for k in range(1, N_DEV):
                pl.semaphore_signal(
                    barrier, inc=1,
                    device_id=((me + k) % N_DEV,),
                    device_id_type=pl.DeviceIdType.MESH)
            pl.semaphore_wait(barrier, N_DEV - 1)

            rs = []
            for k in range(1, N_DEV):
                dst = (me + k) % N_DEV
                rdma = pltpu.make_async_remote_copy(
                    src_ref=accl_ref.at[pl.ds(dst * SLICE, SLICE)],
                    dst_ref=rs_recv.at[k - 1],
                    send_sem=send_sems.at[k - 1],
                    recv_sem=recv_sems.at[k - 1],
                    device_id=(dst,),
                    device_id_type=pl.DeviceIdType.MESH)
                rdma.start()
                rs.append(rdma)
            red = accl_ref[pl.ds(me * SLICE, SLICE), :]
            for k in range(1, N_DEV):
                rs[k - 1].wait_recv()
                red += rs_recv[k - 1]
            for rdma in rs:
                rdma.wait_send()
            accl_ref[pl.ds(me * SLICE, SLICE), :] = red

            ag = []
            for k in range(1, N_DEV):
                rdma = pltpu.make_async_remote_copy(
                    src_ref=accl_ref.at[pl.ds(me * SLICE, SLICE)],
                    dst_ref=ag_recv.at[k - 1],
                    send_sem=send_sems2.at[k - 1],
                    recv_sem=recv_sems2.at[k - 1],
                    device_id=((me + k) % N_DEV,),
                    device_id_type=pl.DeviceIdType.MESH)
                rdma.start()
                ag.append(rdma)
            for k in range(1, N_DEV):
                ag[k - 1].wait_recv()
                src_dev = (me - k) % N_DEV
                accl_ref[pl.ds(src_dev * SLICE, SLICE), :] = ag_recv[k - 1]
            for rdma in ag:
                rdma.wait_send()

            for h in range(HQ):
                lh = _row_to_col(accl_ref[pl.ds(ROWS + h, 1), :])
                t_ref[:, pl.ds(h * DH, DH)] = (
                    accl_ref[pl.ds(h * SQ, SQ), :] / lh)
            stage[2].wait()
            o_ref[...] = jnp.dot(
                t_ref[...].astype(jnp.bfloat16),
                wo_s[...].astype(jnp.bfloat16),
                preferred_element_type=jnp.float32)

    out = pl.pallas_call(
        body,
        grid=(NCHUNKS,),
        in_specs=[
            pl.BlockSpec(memory_space=pl.ANY),
            pl.BlockSpec(memory_space=pl.ANY),
            pl.BlockSpec(memory_space=pl.ANY),
            pl.BlockSpec(memory_space=pl.ANY),
            pl.BlockSpec(memory_space=pl.ANY),
        ],
        out_specs=pl.BlockSpec((SQ, D), lambda j: (0, 0)),
        out_shape=jax.ShapeDtypeStruct((SQ, D), jnp.float32),
        scratch_shapes=[
            pltpu.VMEM((ROWS, DH), jnp.float32),
            pltpu.VMEM((PACK, DH), jnp.float32),
            pltpu.VMEM((ROWS, 1), jnp.float32),
            pltpu.VMEM((N_DEV - 1, SLICE, DH), jnp.float32),
            pltpu.VMEM((N_DEV - 1, SLICE, DH), jnp.float32),
            pltpu.VMEM((SQ, D), jnp.float32),
            pltpu.VMEM((2, 2 * HKV, CHUNK, DH), jnp.float32),
            pltpu.VMEM((SQ, D), jnp.float32),
            pltpu.VMEM((D, D), jnp.float32),
            pltpu.VMEM((D, D), jnp.float32),
            pltpu.SemaphoreType.DMA((2, 2 * HKV)),
            pltpu.SemaphoreType.DMA((3,)),
            pltpu.SemaphoreType.DMA((N_DEV - 1,)),
            pltpu.SemaphoreType.DMA((N_DEV - 1,)),
            pltpu.SemaphoreType.DMA((N_DEV - 1,)),
            pltpu.SemaphoreType.DMA((N_DEV - 1,)),
        ],
        compiler_params=pltpu.CompilerParams(
            dimension_semantics=("arbitrary",),
            collective_id=0 if _COMM else None,
            vmem_limit_bytes=64 * 1024 * 1024,
        ),
    )(x2, Wq, Wo, k3, v3)
    return out.reshape(1, SQ, D)
